# Initial kernel scaffold; baseline (speedup 1.0000x reference)
import jax
import jax.numpy as jnp
from jax.experimental import pallas as pl


def kernel(feature, edge_index, W, b):
    raise NotImplementedError("write your pallas kernel here")


# speedup vs baseline: 8.0997x; 8.0997x over previous
import functools

import jax
import jax.numpy as jnp
from jax import lax
from jax.experimental import pallas as pl
from jax.experimental.pallas import tpu as pltpu
from jax.experimental.pallas import tpu_sc as plsc

N = 10000
D_IN = 128
DH = 64
E = 320000
NC = 2
NS = 16
EPT = E // NS
CH = 80
NCH = EPT // CH
IB = 50
NB = NCH // IB
RPT = 640
RLAST = N - (NS - 1) * RPT
DEGW = 16
RB = 2000

_MESH = plsc.VectorSubcoreMesh(
    core_axis_name="c", subcore_axis_name="s", num_cores=NC, num_subcores=NS
)


@functools.partial(
    pl.kernel,
    out_type=jax.ShapeDtypeStruct((NC, N, DEGW), jnp.float32),
    mesh=_MESH,
    scratch_types=[
        pltpu.VMEM_SHARED((N, DEGW), jnp.float32),
        pltpu.VMEM((IB, CH), jnp.int32),
        pltpu.VMEM((CH, DEGW), jnp.float32),
        pltpu.VMEM((CH, DEGW), jnp.float32),
    ],
)
def _sc_degrees(idx_hbm, deg_hbm, deg_sh, idx_v, zero_v, ones_v):
    c = lax.axis_index("c")
    s = lax.axis_index("s")
    zv = jnp.zeros((16,), jnp.float32)
    ov = jnp.ones((16,), jnp.float32)

    def fill(i, _):
        zero_v[i, :] = zv
        ones_v[i, :] = ov
        return 0

    lax.fori_loop(0, CH, fill, 0)

    @pl.when(s < NS - 1)
    def _():
        for k in range(RPT // CH):
            pltpu.sync_copy(zero_v, deg_sh.at[pl.ds(s * RPT + k * CH, CH)])

    @pl.when(s == NS - 1)
    def _():
        for k in range(RLAST // CH):
            pltpu.sync_copy(zero_v, deg_sh.at[pl.ds((NS - 1) * RPT + k * CH, CH)])

    plsc.subcore_barrier()

    def batch(bi, _):
        pltpu.sync_copy(idx_hbm.at[c, s, bi], idx_v)

        def step(j, _):
            pltpu.sync_copy(ones_v, deg_sh.at[idx_v.at[j]], add=True)
            return 0

        lax.fori_loop(0, IB, step, 0)
        return 0

    lax.fori_loop(0, NB, batch, 0)
    plsc.subcore_barrier()

    @pl.when(s < NS - 1)
    def _():
        pltpu.sync_copy(
            deg_sh.at[pl.ds(s * RPT, RPT)], deg_hbm.at[c, pl.ds(s * RPT, RPT)]
        )

    @pl.when(s == NS - 1)
    def _():
        pltpu.sync_copy(
            deg_sh.at[pl.ds((NS - 1) * RPT, RLAST)],
            deg_hbm.at[c, pl.ds((NS - 1) * RPT, RLAST)],
        )


@functools.partial(
    pl.kernel,
    out_type=jax.ShapeDtypeStruct((NC, N, DH), jnp.float32),
    mesh=_MESH,
    scratch_types=[
        pltpu.VMEM_SHARED((N, DH), jnp.float32),
        pltpu.VMEM_SHARED((N, DH), jnp.float32),
        pltpu.VMEM((IB, CH), jnp.int32),
        pltpu.VMEM((IB, CH), jnp.int32),
        pltpu.VMEM((CH, DH), jnp.float32),
        pltpu.SemaphoreType.DMA,
    ],
)
def _sc_scatter(
    h_hbm, src_hbm, dst_hbm, agg_hbm, h_sh, agg_sh, src_v, dst_v, msg_v, sem
):
    c = lax.axis_index("c")
    s = lax.axis_index("s")
    zv = jnp.zeros((16,), jnp.float32)

    def fill_zero(i, _):
        for k in range(DH // 16):
            msg_v[i, pl.ds(k * 16, 16)] = zv
        return 0

    lax.fori_loop(0, CH, fill_zero, 0)

    @pl.when(s < NS - 1)
    def _():
        for k in range(RPT // CH):
            pltpu.sync_copy(msg_v, agg_sh.at[pl.ds(s * RPT + k * CH, CH)])
        pltpu.sync_copy(
            h_hbm.at[c, pl.ds(s * RPT, RPT)], h_sh.at[pl.ds(s * RPT, RPT)]
        )

    @pl.when(s == NS - 1)
    def _():
        base = (NS - 1) * RPT
        for k in range(RLAST // CH):
            pltpu.sync_copy(msg_v, agg_sh.at[pl.ds(base + k * CH, CH)])
        pltpu.sync_copy(
            h_hbm.at[c, pl.ds(base, RLAST)], h_sh.at[pl.ds(base, RLAST)]
        )

    plsc.subcore_barrier()

    def batch(bi, _):
        pltpu.sync_copy(src_hbm.at[s, bi], src_v)
        pltpu.sync_copy(dst_hbm.at[s, bi], dst_v)

        def step(j, _):
            pltpu.async_copy(h_sh.at[src_v.at[j]], msg_v, sem).wait()
            pltpu.sync_copy(msg_v, agg_sh.at[dst_v.at[j]], add=True)
            return 0

        lax.fori_loop(0, IB, step, 0)
        return 0

    lax.fori_loop(0, NB, batch, 0)
    plsc.subcore_barrier()

    @pl.when(s < NS - 1)
    def _():
        pltpu.sync_copy(
            agg_sh.at[pl.ds(s * RPT, RPT)], agg_hbm.at[c, pl.ds(s * RPT, RPT)]
        )

    @pl.when(s == NS - 1)
    def _():
        base = (NS - 1) * RPT
        pltpu.sync_copy(
            agg_sh.at[pl.ds(base, RLAST)], agg_hbm.at[c, pl.ds(base, RLAST)]
        )


def _tc_prepare_body(f_ref, w_ref, deg_ref, out_ref):
    h = jnp.dot(f_ref[...], w_ref[0], preferred_element_type=jnp.float32)
    norm = lax.rsqrt(jnp.maximum(deg_ref[:, 0:1], 1.0))
    out_ref[...] = (h * norm)[None]


_tc_prepare = pl.pallas_call(
    _tc_prepare_body,
    grid=(N // RB, NC),
    in_specs=[
        pl.BlockSpec((RB, D_IN), lambda i, j: (i, 0)),
        pl.BlockSpec((1, D_IN, DH), lambda i, j: (j, 0, 0)),
        pl.BlockSpec((RB, DEGW), lambda i, j: (i, 0)),
    ],
    out_specs=pl.BlockSpec((1, RB, DH), lambda i, j: (j, i, 0)),
    out_shape=jax.ShapeDtypeStruct((NC, N, DH), jnp.float32),
)


def _tc_finish_body(agg_ref, deg_ref, b_ref, out_ref):
    norm = lax.rsqrt(jnp.maximum(deg_ref[:, 0:1], 1.0))
    agg = jnp.concatenate([agg_ref[0], agg_ref[1]], axis=1)
    out_ref[...] = jnp.maximum(agg * norm + b_ref[...], 0.0)


_tc_finish = pl.pallas_call(
    _tc_finish_body,
    grid=(N // RB,),
    in_specs=[
        pl.BlockSpec((NC, RB, DH), lambda i: (0, i, 0)),
        pl.BlockSpec((RB, DEGW), lambda i: (i, 0)),
        pl.BlockSpec((1, D_IN), lambda i: (0, 0)),
    ],
    out_specs=pl.BlockSpec((RB, D_IN), lambda i: (i, 0)),
    out_shape=jax.ShapeDtypeStruct((N, D_IN), jnp.float32),
)


def kernel(feature, edge_index, W, b):
    idx5 = edge_index.reshape(2, NS, NB, IB, CH)
    w2 = W.reshape(D_IN, NC, DH).transpose(1, 0, 2)
    deg16 = _sc_degrees(idx5)
    h2 = _tc_prepare(feature, w2, deg16[0])
    agg2 = _sc_scatter(h2, idx5[0], idx5[1])
    return _tc_finish(agg2, deg16[1], b.reshape(1, D_IN))


# speedup vs baseline: 8.1712x; 1.0088x over previous
import functools

import jax
import jax.numpy as jnp
from jax import lax
from jax.experimental import pallas as pl
from jax.experimental.pallas import tpu as pltpu
from jax.experimental.pallas import tpu_sc as plsc

N = 10000
D_IN = 128
DH = 64
E = 320000
NC = 2
NS = 16
EPT = E // NS
CH = 80
NCH = EPT // CH
IB = 50
NB = NCH // IB
RPT = 640
RLAST = N - (NS - 1) * RPT
DEGW = 16
RB = 2000

_MESH = plsc.VectorSubcoreMesh(
    core_axis_name="c", subcore_axis_name="s", num_cores=NC, num_subcores=NS
)


@functools.partial(
    pl.kernel,
    out_type=jax.ShapeDtypeStruct((NC, N, DEGW), jnp.float32),
    mesh=_MESH,
    scratch_types=[
        pltpu.VMEM_SHARED((N, DEGW), jnp.float32),
        pltpu.VMEM((IB, CH), jnp.int32),
        pltpu.VMEM((CH, DEGW), jnp.float32),
        pltpu.VMEM((CH, DEGW), jnp.float32),
        pltpu.SemaphoreType.DMA,
    ],
)
def _sc_degrees(idx_hbm, deg_hbm, deg_sh, idx_v, zero_v, ones_v, sem):
    c = lax.axis_index("c")
    s = lax.axis_index("s")
    zv = jnp.zeros((16,), jnp.float32)
    ov = jnp.ones((16,), jnp.float32)

    def fill(i, _):
        zero_v[i, :] = zv
        ones_v[i, :] = ov
        return 0

    lax.fori_loop(0, CH, fill, 0)

    @pl.when(s < NS - 1)
    def _():
        for k in range(RPT // CH):
            pltpu.sync_copy(zero_v, deg_sh.at[pl.ds(s * RPT + k * CH, CH)])

    @pl.when(s == NS - 1)
    def _():
        for k in range(RLAST // CH):
            pltpu.sync_copy(zero_v, deg_sh.at[pl.ds((NS - 1) * RPT + k * CH, CH)])

    plsc.subcore_barrier()

    def batch(bi, _):
        pltpu.sync_copy(idx_hbm.at[c, s, bi], idx_v)

        def fire(j, _):
            pltpu.async_copy(ones_v, deg_sh.at[idx_v.at[j]], sem, add=True)
            return 0

        lax.fori_loop(0, IB, fire, 0)

        def drain(j, _):
            pltpu.make_async_copy(ones_v, deg_sh.at[idx_v.at[j]], sem).wait()
            return 0

        lax.fori_loop(0, IB, drain, 0)
        return 0

    lax.fori_loop(0, NB, batch, 0)
    plsc.subcore_barrier()

    @pl.when(s < NS - 1)
    def _():
        pltpu.sync_copy(
            deg_sh.at[pl.ds(s * RPT, RPT)], deg_hbm.at[c, pl.ds(s * RPT, RPT)]
        )

    @pl.when(s == NS - 1)
    def _():
        pltpu.sync_copy(
            deg_sh.at[pl.ds((NS - 1) * RPT, RLAST)],
            deg_hbm.at[c, pl.ds((NS - 1) * RPT, RLAST)],
        )


@functools.partial(
    pl.kernel,
    out_type=jax.ShapeDtypeStruct((NC, N, DH), jnp.float32),
    mesh=_MESH,
    scratch_types=[
        pltpu.VMEM_SHARED((N, DH), jnp.float32),
        pltpu.VMEM_SHARED((N, DH), jnp.float32),
        pltpu.VMEM((IB, CH), jnp.int32),
        pltpu.VMEM((IB, CH), jnp.int32),
        pltpu.VMEM((CH, DH), jnp.float32),
        pltpu.VMEM((CH, DH), jnp.float32),
        pltpu.SemaphoreType.DMA,
        pltpu.SemaphoreType.DMA,
        pltpu.SemaphoreType.DMA,
        pltpu.SemaphoreType.DMA,
    ],
)
def _sc_scatter(
    h_hbm, src_hbm, dst_hbm, agg_hbm, h_sh, agg_sh, src_v, dst_v,
    msg_v, msg_w, sem_g0, sem_g1, sem_s0, sem_s1
):
    c = lax.axis_index("c")
    s = lax.axis_index("s")
    zv = jnp.zeros((16,), jnp.float32)

    def fill_zero(i, _):
        for k in range(DH // 16):
            msg_v[i, pl.ds(k * 16, 16)] = zv
        return 0

    lax.fori_loop(0, CH, fill_zero, 0)

    @pl.when(s < NS - 1)
    def _():
        for k in range(RPT // CH):
            pltpu.sync_copy(msg_v, agg_sh.at[pl.ds(s * RPT + k * CH, CH)])
        pltpu.sync_copy(
            h_hbm.at[c, pl.ds(s * RPT, RPT)], h_sh.at[pl.ds(s * RPT, RPT)]
        )

    @pl.when(s == NS - 1)
    def _():
        base = (NS - 1) * RPT
        for k in range(RLAST // CH):
            pltpu.sync_copy(msg_v, agg_sh.at[pl.ds(base + k * CH, CH)])
        pltpu.sync_copy(
            h_hbm.at[c, pl.ds(base, RLAST)], h_sh.at[pl.ds(base, RLAST)]
        )

    plsc.subcore_barrier()

    def batch(bi, _):
        pltpu.sync_copy(src_hbm.at[s, bi], src_v)
        pltpu.sync_copy(dst_hbm.at[s, bi], dst_v)

        def pair(p, _):
            j0 = 2 * p
            j1 = j0 + 1
            g0 = pltpu.async_copy(h_sh.at[src_v.at[j0]], msg_v, sem_g0)
            g1 = pltpu.async_copy(h_sh.at[src_v.at[j1]], msg_w, sem_g1)
            g0.wait()
            s0 = pltpu.async_copy(msg_v, agg_sh.at[dst_v.at[j0]], sem_s0, add=True)
            g1.wait()
            s1 = pltpu.async_copy(msg_w, agg_sh.at[dst_v.at[j1]], sem_s1, add=True)
            s0.wait()
            s1.wait()
            return 0

        lax.fori_loop(0, IB // 2, pair, 0)
        return 0

    lax.fori_loop(0, NB, batch, 0)
    plsc.subcore_barrier()

    @pl.when(s < NS - 1)
    def _():
        pltpu.sync_copy(
            agg_sh.at[pl.ds(s * RPT, RPT)], agg_hbm.at[c, pl.ds(s * RPT, RPT)]
        )

    @pl.when(s == NS - 1)
    def _():
        base = (NS - 1) * RPT
        pltpu.sync_copy(
            agg_sh.at[pl.ds(base, RLAST)], agg_hbm.at[c, pl.ds(base, RLAST)]
        )


def _tc_prepare_body(f_ref, w_ref, deg_ref, out_ref):
    h = jnp.dot(f_ref[...], w_ref[0], preferred_element_type=jnp.float32)
    norm = lax.rsqrt(jnp.maximum(deg_ref[:, 0:1], 1.0))
    out_ref[...] = (h * norm)[None]


_tc_prepare = pl.pallas_call(
    _tc_prepare_body,
    grid=(N // RB, NC),
    in_specs=[
        pl.BlockSpec((RB, D_IN), lambda i, j: (i, 0)),
        pl.BlockSpec((1, D_IN, DH), lambda i, j: (j, 0, 0)),
        pl.BlockSpec((RB, DEGW), lambda i, j: (i, 0)),
    ],
    out_specs=pl.BlockSpec((1, RB, DH), lambda i, j: (j, i, 0)),
    out_shape=jax.ShapeDtypeStruct((NC, N, DH), jnp.float32),
)


def _tc_finish_body(agg_ref, deg_ref, b_ref, out_ref):
    norm = lax.rsqrt(jnp.maximum(deg_ref[:, 0:1], 1.0))
    agg = jnp.concatenate([agg_ref[0], agg_ref[1]], axis=1)
    out_ref[...] = jnp.maximum(agg * norm + b_ref[...], 0.0)


_tc_finish = pl.pallas_call(
    _tc_finish_body,
    grid=(N // RB,),
    in_specs=[
        pl.BlockSpec((NC, RB, DH), lambda i: (0, i, 0)),
        pl.BlockSpec((RB, DEGW), lambda i: (i, 0)),
        pl.BlockSpec((1, D_IN), lambda i: (0, 0)),
    ],
    out_specs=pl.BlockSpec((RB, D_IN), lambda i: (i, 0)),
    out_shape=jax.ShapeDtypeStruct((N, D_IN), jnp.float32),
)


def kernel(feature, edge_index, W, b):
    idx5 = edge_index.reshape(2, NS, NB, IB, CH)
    w2 = W.reshape(D_IN, NC, DH).transpose(1, 0, 2)
    deg16 = _sc_degrees(idx5)
    h2 = _tc_prepare(feature, w2, deg16[0])
    agg2 = _sc_scatter(h2, idx5[0], idx5[1])
    return _tc_finish(agg2, deg16[1], b.reshape(1, D_IN))
